# baseline (speedup 1.0000x reference)
import functools

import jax
import jax.numpy as jnp
from jax import lax
from jax.experimental import pallas as pl
from jax.experimental.pallas import tpu as pltpu
from jax.experimental.pallas import tpu_sc as plsc

_N = 10000
_E = 320000
_D = 128
_H = 32
_C = 10
_G = 16

_F = 128 // _H
_NR = _N // _F

_NC = 2
_NS = 16
_NW = _NC * _NS
_EPW = _E // _NW
_K = 50
_NCH = _EPW // _K
_U = 25
_NG = _NCH // _U
_NP = 10240
_RPS = _NP // _NS

_BN_SCALE = float(1.0 / (1.0 + 1e-5) ** 0.5)


_sc_mesh = plsc.VectorSubcoreMesh(core_axis_name="c", subcore_axis_name="s")


@functools.partial(
    pl.kernel,
    out_type=jax.ShapeDtypeStruct((_NC, _NP, _H), jnp.float32),
    mesh=_sc_mesh,
    scratch_types=[
        pltpu.VMEM((_NCH, _K), jnp.int32),
        pltpu.VMEM((_NCH, _K), jnp.int32),
        pltpu.VMEM((2, _U, _K, _H), jnp.float32),
        pltpu.VMEM_SHARED((_NP, _H), jnp.float32),
        pltpu.SemaphoreType.DMA,
        pltpu.SemaphoreType.DMA,
    ],
    compiler_params=pltpu.CompilerParams(use_tc_tiling_on_sc=False),
)
def _sc_segsum(p_hbm, e4_hbm, zeros_hbm, out_hbm,
               src_v, dst_v, rows_v, acc_sh, gsem, ssem):
    c = lax.axis_index("c")
    s = lax.axis_index("s")
    wid = c * _NS + s
    zd = pltpu.async_copy(zeros_hbm.at[pl.ds(s * _RPS, _RPS)],
                          acc_sh.at[pl.ds(s * _RPS, _RPS)], gsem)
    sd0 = pltpu.async_copy(e4_hbm.at[0, wid], src_v, gsem)
    dd0 = pltpu.async_copy(e4_hbm.at[1, wid], dst_v, gsem)
    zd.wait()
    sd0.wait()
    dd0.wait()
    plsc.subcore_barrier()

    def _fire_gathers(g, t):
        for u in range(_U):
            pltpu.async_copy(p_hbm.at[src_v.at[g * _U + u]],
                             rows_v.at[t, u], gsem)

    def _drain(sem):
        pltpu.make_async_copy(zeros_hbm.at[pl.ds(0, _K)],
                              rows_v.at[0, 0], sem).wait()

    def _process(g, t):
        for u in range(_U):
            _drain(gsem)
            pltpu.async_copy(rows_v.at[t, u],
                             acc_sh.at[dst_v.at[g * _U + u]], ssem, add=True)

    _fire_gathers(0, 0)
    _fire_gathers(1, 1)
    _process(0, 0)

    def body(g, carry):
        t = lax.rem(g, 2)
        for _ in range(_U):
            _drain(ssem)
        _fire_gathers(g + 1, 1 - t)
        _process(g, t)
        return carry

    lax.fori_loop(1, _NG - 1, body, 0)
    for _ in range(_U):
        _drain(ssem)
    _process(_NG - 1, (_NG - 1) % 2)
    for _ in range(_U):
        _drain(ssem)
    plsc.subcore_barrier()
    pltpu.sync_copy(acc_sh.at[pl.ds(s * _RPS, _RPS)],
                    out_hbm.at[c, pl.ds(s * _RPS, _RPS)])


def _proj_body(x_ref, w_ref, o_ref):
    o_ref[...] = jnp.dot(x_ref[...], w_ref[...],
                         preferred_element_type=jnp.float32)


def _project(x, w):
    return pl.pallas_call(
        _proj_body,
        out_shape=jax.ShapeDtypeStruct((_N, _H), jnp.float32),
    )(x, w)


def _blockdiag(w):
    row = jnp.concatenate([w, w, w, w], axis=0)
    wt = jnp.concatenate([row, row, row, row], axis=1)
    blk = (lax.broadcasted_iota(jnp.int32, (128, 128), 0) // _H ==
           lax.broadcasted_iota(jnp.int32, (128, 128), 1) // _H)
    return wt * blk.astype(jnp.float32)


def _tile4(v):
    return jnp.concatenate([v, v, v, v], axis=1)


def _mid_body(p_ref, part_ref, b1_ref, w2_ref, b2_ref, g_ref, b_ref,
              w1n_ref, o_ref):
    agg = part_ref[0, :_NR, :] + part_ref[1, :_NR, :]
    z1 = jnp.maximum(p_ref[...] + agg + _tile4(b1_ref[...]), 0.0)
    z = jnp.maximum(
        jnp.dot(z1, _blockdiag(w2_ref[...]),
                preferred_element_type=jnp.float32)
        + _tile4(b2_ref[...]), 0.0)
    h = z * (_BN_SCALE * _tile4(g_ref[...])) + _tile4(b_ref[...])
    o_ref[...] = jnp.dot(h, _blockdiag(w1n_ref[...]),
                         preferred_element_type=jnp.float32)


def _mid(p, parts_p, b1, w2, b2, g, b, w1n):
    return pl.pallas_call(
        _mid_body,
        out_shape=jax.ShapeDtypeStruct((_NR, 128), jnp.float32),
    )(p, parts_p, b1.reshape(1, _H), w2, b2.reshape(1, _H),
      g.reshape(1, _H), b.reshape(1, _H), w1n)


def _final_body(p_ref, part_ref, b1_ref, w2_ref, b2_ref, g_ref, b_ref,
                batch_ref, fc1w_ref, fc1b_ref, fc2w_ref, fc2b_ref, o_ref):
    agg = part_ref[0, :_NR, :] + part_ref[1, :_NR, :]
    z1 = jnp.maximum(p_ref[...] + agg + _tile4(b1_ref[...]), 0.0)
    z = jnp.maximum(
        jnp.dot(z1, _blockdiag(w2_ref[...]),
                preferred_element_type=jnp.float32)
        + _tile4(b2_ref[...]), 0.0)
    h = z * (_BN_SCALE * _tile4(g_ref[...])) + _tile4(b_ref[...])
    bt = batch_ref[...]
    sums = jnp.zeros((_G, _H), dtype=jnp.float32)
    cnts = jnp.zeros((_G, 1), dtype=jnp.float32)
    for j in range(_F):
        oh = (bt[j:j + 1, :] == lax.broadcasted_iota(jnp.int32, (_G, _NR), 0))
        oh = oh.astype(jnp.float32)
        sums = sums + jnp.dot(oh, h[:, _H * j:_H * (j + 1)],
                              preferred_element_type=jnp.float32)
        cnts = cnts + jnp.sum(oh, axis=1, keepdims=True)
    pooled = sums / jnp.maximum(cnts, 1.0)
    a = jnp.maximum(
        jnp.dot(pooled, fc1w_ref[...], preferred_element_type=jnp.float32)
        + fc1b_ref[...], 0.0)
    logits = jnp.dot(a, fc2w_ref[...], preferred_element_type=jnp.float32) \
        + fc2b_ref[...]
    m = jnp.max(logits, axis=-1, keepdims=True)
    lse = jnp.log(jnp.sum(jnp.exp(logits - m), axis=-1, keepdims=True)) + m
    o_ref[...] = logits - lse


def _final(p, parts_p, b1, w2, b2, g, b, batch_t,
           fc1_w, fc1_b, fc2_w, fc2_b):
    return pl.pallas_call(
        _final_body,
        out_shape=jax.ShapeDtypeStruct((_G, _C), jnp.float32),
    )(p, parts_p, b1.reshape(1, _H), w2, b2.reshape(1, _H),
      g.reshape(1, _H), b.reshape(1, _H), batch_t,
      fc1_w, fc1_b.reshape(1, _H), fc2_w, fc2_b.reshape(1, _C))


def kernel(x, edge_index, batch,
           conv0_w1, conv0_b1, conv0_w2, conv0_b2, bn0_g, bn0_b,
           conv1_w1, conv1_b1, conv1_w2, conv1_b2, bn1_g, bn1_b,
           conv2_w1, conv2_b1, conv2_w2, conv2_b2, bn2_g, bn2_b,
           fc1_w, fc1_b, fc2_w, fc2_b):
    e4 = edge_index.reshape(2, _NW, _NCH, _K)
    batch_t = batch.reshape(_NR, _F).T
    zeros = jnp.zeros((_NP, _H), dtype=jnp.float32)

    p0 = _project(x, conv0_w1)
    parts0 = _sc_segsum(p0, e4, zeros)
    p1 = _mid(p0.reshape(_NR, 128), parts0.reshape(_NC, _NP // _F, 128),
              conv0_b1, conv0_w2, conv0_b2, bn0_g, bn0_b, conv1_w1)
    parts1 = _sc_segsum(p1.reshape(_N, _H), e4, zeros)
    p2 = _mid(p1, parts1.reshape(_NC, _NP // _F, 128),
              conv1_b1, conv1_w2, conv1_b2, bn1_g, bn1_b, conv2_w1)
    parts2 = _sc_segsum(p2.reshape(_N, _H), e4, zeros)
    return _final(p2, parts2.reshape(_NC, _NP // _F, 128),
                  conv2_b1, conv2_w2, conv2_b2, bn2_g, bn2_b,
                  batch_t, fc1_w, fc1_b, fc2_w, fc2_b)


# speedup vs baseline: 1.1198x; 1.1198x over previous
import functools

import jax
import jax.numpy as jnp
from jax import lax
from jax.experimental import pallas as pl
from jax.experimental.pallas import tpu as pltpu
from jax.experimental.pallas import tpu_sc as plsc

_N = 10000
_E = 320000
_D = 128
_H = 32
_C = 10
_G = 16

_F = 128 // _H
_NR = _N // _F

_NC = 2
_NS = 16
_NW = _NC * _NS
_EPW = _E // _NW
_K = 80
_NCH = _EPW // _K
_U = 25
_NP = 10240
_RPS = _NP // _NS

_BN_SCALE = float(1.0 / (1.0 + 1e-5) ** 0.5)


_sc_mesh = plsc.VectorSubcoreMesh(core_axis_name="c", subcore_axis_name="s")


@functools.partial(
    pl.kernel,
    out_type=jax.ShapeDtypeStruct((_NC, _NP, _H), jnp.float32),
    mesh=_sc_mesh,
    scratch_types=[
        pltpu.VMEM((_NCH, _K), jnp.int32),
        pltpu.VMEM((_NCH, _K), jnp.int32),
        pltpu.VMEM((_U, _K, _H), jnp.float32),
        pltpu.VMEM_SHARED((_NP, _H), jnp.float32),
        pltpu.SemaphoreType.DMA,
        pltpu.SemaphoreType.DMA,
    ],
    compiler_params=pltpu.CompilerParams(use_tc_tiling_on_sc=False),
)
def _sc_segsum(p_hbm, e4_hbm, zeros_hbm, out_hbm,
               src_v, dst_v, rows_v, acc_sh, gsem, ssem):
    c = lax.axis_index("c")
    s = lax.axis_index("s")
    wid = c * _NS + s
    zd = pltpu.async_copy(zeros_hbm.at[pl.ds(s * _RPS, _RPS)],
                          acc_sh.at[pl.ds(s * _RPS, _RPS)], gsem)
    sd0 = pltpu.async_copy(e4_hbm.at[0, wid], src_v, gsem)
    dd0 = pltpu.async_copy(e4_hbm.at[1, wid], dst_v, gsem)
    zd.wait()
    sd0.wait()
    dd0.wait()
    plsc.subcore_barrier()

    def body(i, carry):
        gd = [pltpu.async_copy(p_hbm.at[src_v.at[i * _U + u]],
                               rows_v.at[u], gsem)
              for u in range(_U)]
        sd = []
        for u in range(_U):
            gd[u].wait()
            sd.append(pltpu.async_copy(rows_v.at[u],
                                       acc_sh.at[dst_v.at[i * _U + u]],
                                       ssem, add=True))
        for u in range(_U):
            sd[u].wait()
        return carry

    lax.fori_loop(0, _NCH // _U, body, 0)
    plsc.subcore_barrier()
    pltpu.sync_copy(acc_sh.at[pl.ds(s * _RPS, _RPS)],
                    out_hbm.at[c, pl.ds(s * _RPS, _RPS)])


def _proj_body(x_ref, w_ref, o_ref):
    o_ref[...] = jnp.dot(x_ref[...], w_ref[...],
                         preferred_element_type=jnp.float32)


def _project(x, w):
    return pl.pallas_call(
        _proj_body,
        out_shape=jax.ShapeDtypeStruct((_N, _H), jnp.float32),
    )(x, w)


def _blockdiag(w):
    row = jnp.concatenate([w, w, w, w], axis=0)
    wt = jnp.concatenate([row, row, row, row], axis=1)
    blk = (lax.broadcasted_iota(jnp.int32, (128, 128), 0) // _H ==
           lax.broadcasted_iota(jnp.int32, (128, 128), 1) // _H)
    return wt * blk.astype(jnp.float32)


def _tile4(v):
    return jnp.concatenate([v, v, v, v], axis=1)


def _mid_body(p_ref, part_ref, b1_ref, w2_ref, b2_ref, g_ref, b_ref,
              w1n_ref, o_ref):
    agg = part_ref[0, :_NR, :] + part_ref[1, :_NR, :]
    z1 = jnp.maximum(p_ref[...] + agg + _tile4(b1_ref[...]), 0.0)
    z = jnp.maximum(
        jnp.dot(z1, _blockdiag(w2_ref[...]),
                preferred_element_type=jnp.float32)
        + _tile4(b2_ref[...]), 0.0)
    h = z * (_BN_SCALE * _tile4(g_ref[...])) + _tile4(b_ref[...])
    o_ref[...] = jnp.dot(h, _blockdiag(w1n_ref[...]),
                         preferred_element_type=jnp.float32)


def _mid(p, parts_p, b1, w2, b2, g, b, w1n):
    return pl.pallas_call(
        _mid_body,
        out_shape=jax.ShapeDtypeStruct((_NR, 128), jnp.float32),
    )(p, parts_p, b1.reshape(1, _H), w2, b2.reshape(1, _H),
      g.reshape(1, _H), b.reshape(1, _H), w1n)


def _final_body(p_ref, part_ref, b1_ref, w2_ref, b2_ref, g_ref, b_ref,
                batch_ref, fc1w_ref, fc1b_ref, fc2w_ref, fc2b_ref, o_ref):
    agg = part_ref[0, :_NR, :] + part_ref[1, :_NR, :]
    z1 = jnp.maximum(p_ref[...] + agg + _tile4(b1_ref[...]), 0.0)
    z = jnp.maximum(
        jnp.dot(z1, _blockdiag(w2_ref[...]),
                preferred_element_type=jnp.float32)
        + _tile4(b2_ref[...]), 0.0)
    h = z * (_BN_SCALE * _tile4(g_ref[...])) + _tile4(b_ref[...])
    bt = batch_ref[...]
    sums = jnp.zeros((_G, _H), dtype=jnp.float32)
    cnts = jnp.zeros((_G, 1), dtype=jnp.float32)
    for j in range(_F):
        oh = (bt[j:j + 1, :] == lax.broadcasted_iota(jnp.int32, (_G, _NR), 0))
        oh = oh.astype(jnp.float32)
        sums = sums + jnp.dot(oh, h[:, _H * j:_H * (j + 1)],
                              preferred_element_type=jnp.float32)
        cnts = cnts + jnp.sum(oh, axis=1, keepdims=True)
    pooled = sums / jnp.maximum(cnts, 1.0)
    a = jnp.maximum(
        jnp.dot(pooled, fc1w_ref[...], preferred_element_type=jnp.float32)
        + fc1b_ref[...], 0.0)
    logits = jnp.dot(a, fc2w_ref[...], preferred_element_type=jnp.float32) \
        + fc2b_ref[...]
    m = jnp.max(logits, axis=-1, keepdims=True)
    lse = jnp.log(jnp.sum(jnp.exp(logits - m), axis=-1, keepdims=True)) + m
    o_ref[...] = logits - lse


def _final(p, parts_p, b1, w2, b2, g, b, batch_t,
           fc1_w, fc1_b, fc2_w, fc2_b):
    return pl.pallas_call(
        _final_body,
        out_shape=jax.ShapeDtypeStruct((_G, _C), jnp.float32),
    )(p, parts_p, b1.reshape(1, _H), w2, b2.reshape(1, _H),
      g.reshape(1, _H), b.reshape(1, _H), batch_t,
      fc1_w, fc1_b.reshape(1, _H), fc2_w, fc2_b.reshape(1, _C))


def kernel(x, edge_index, batch,
           conv0_w1, conv0_b1, conv0_w2, conv0_b2, bn0_g, bn0_b,
           conv1_w1, conv1_b1, conv1_w2, conv1_b2, bn1_g, bn1_b,
           conv2_w1, conv2_b1, conv2_w2, conv2_b2, bn2_g, bn2_b,
           fc1_w, fc1_b, fc2_w, fc2_b):
    e4 = edge_index.reshape(2, _NW, _NCH, _K)
    batch_t = batch.reshape(_NR, _F).T
    zeros = jnp.zeros((_NP, _H), dtype=jnp.float32)

    p0 = _project(x, conv0_w1)
    parts0 = _sc_segsum(p0, e4, zeros)
    p1 = _mid(p0.reshape(_NR, 128), parts0.reshape(_NC, _NP // _F, 128),
              conv0_b1, conv0_w2, conv0_b2, bn0_g, bn0_b, conv1_w1)
    parts1 = _sc_segsum(p1.reshape(_N, _H), e4, zeros)
    p2 = _mid(p1, parts1.reshape(_NC, _NP // _F, 128),
              conv1_b1, conv1_w2, conv1_b2, bn1_g, bn1_b, conv2_w1)
    parts2 = _sc_segsum(p2.reshape(_N, _H), e4, zeros)
    return _final(p2, parts2.reshape(_NC, _NP // _F, 128),
                  conv2_b1, conv2_w2, conv2_b2, bn2_g, bn2_b,
                  batch_t, fc1_w, fc1_b, fc2_w, fc2_b)
